# Initial kernel scaffold; baseline (speedup 1.0000x reference)
import jax
import jax.numpy as jnp
from jax.experimental import pallas as pl


def kernel(x, edge_index):
    raise NotImplementedError("write your pallas kernel here")


# speedup vs baseline: 3.6706x; 3.6706x over previous
import functools

import jax
import jax.numpy as jnp
from jax import lax
from jax.experimental import pallas as pl
from jax.experimental.pallas import tpu as pltpu
from jax.experimental.pallas import tpu_sc as plsc

N_NODES = 10000
N_EDGES = 160000
D_FEAT = 256
DH = 128
NC = 2
NS = 16
L = 16
E_PER_TILE = N_EDGES // NS
BATCH = 80
NB = E_PER_TILE // BATCH
ROWS_PER_TILE = N_NODES // NS
ZROWS = 25


def _mp_body(x2, src, dst, out, idx_raw, idx_d, idx_g, rows, zbuf, acc, sem):
    cid = lax.axis_index("c")
    sid = lax.axis_index("s")

    zero = jnp.zeros((L,), jnp.float32)

    def zfill(i, _):
        r = i // (DH // L)
        c = (i % (DH // L)) * L
        zbuf[r, pl.ds(c, L)] = zero
        return 0

    lax.fori_loop(0, ZROWS * (DH // L), zfill, 0)

    def zcopy(k, _):
        pltpu.sync_copy(zbuf, acc.at[pl.ds(sid * ROWS_PER_TILE + k * ZROWS, ZROWS)])
        return 0

    lax.fori_loop(0, ROWS_PER_TILE // ZROWS, zcopy, 0)
    plsc.subcore_barrier()

    def body(b, _):
        base = sid * E_PER_TILE + b * BATCH
        pltpu.sync_copy(src.at[pl.ds(base, BATCH)], idx_raw)
        pltpu.sync_copy(dst.at[pl.ds(base, BATCH)], idx_d)

        def tfm(i, _):
            v = idx_raw[pl.ds(i * L, L)]
            idx_g[pl.ds(i * L, L)] = v * 2 + cid
            return 0

        lax.fori_loop(0, BATCH // L, tfm, 0)
        pltpu.async_copy(x2.at[idx_g], rows, sem).wait()
        pltpu.sync_copy(rows, acc.at[idx_d], add=True)
        return 0

    lax.fori_loop(0, NB, body, 0)
    plsc.subcore_barrier()

    r0 = sid * ROWS_PER_TILE
    pltpu.sync_copy(acc.at[pl.ds(r0, ROWS_PER_TILE)],
                    out.at[pl.ds(r0, ROWS_PER_TILE), cid])


_mp_kernel = functools.partial(
    pl.kernel,
    mesh=plsc.VectorSubcoreMesh(core_axis_name="c", subcore_axis_name="s"),
    out_type=jax.ShapeDtypeStruct((N_NODES, NC, DH), jnp.float32),
    scratch_types=[
        pltpu.VMEM((BATCH,), jnp.int32),
        pltpu.VMEM((BATCH,), jnp.int32),
        pltpu.VMEM((BATCH,), jnp.int32),
        pltpu.VMEM((BATCH, DH), jnp.float32),
        pltpu.VMEM((ZROWS, DH), jnp.float32),
        pltpu.VMEM_SHARED((N_NODES, DH), jnp.float32),
        pltpu.SemaphoreType.DMA,
    ],
)(_mp_body)


@jax.jit
def kernel(x, edge_index):
    x2 = x.reshape(N_NODES * 2, DH)
    out = _mp_kernel(x2, edge_index[1], edge_index[0])
    return out.reshape(N_NODES, D_FEAT)


# speedup vs baseline: 8.1398x; 2.2176x over previous
import functools

import jax
import jax.numpy as jnp
from jax import lax
from jax.experimental import pallas as pl
from jax.experimental.pallas import tpu as pltpu
from jax.experimental.pallas import tpu_sc as plsc

N_NODES = 10000
N_EDGES = 160000
D_FEAT = 256
DH = 128
NC = 2
NS = 16
L = 16
PAD_ROWS = 8
E_PAD = 163840
E_PER_TILE = E_PAD // NS
BATCH = 64
NB = E_PER_TILE // BATCH
NBUF = 4
ROWS_PER_TILE = N_NODES // NS
ZROWS = 25


def _mp_body(src4, dst4, x2, out, srcb, dstb, rows, zbuf, acc, *sems):
    gsems, isems = sems[:NBUF], sems[NBUF:]
    cid = lax.axis_index("c")
    sid = lax.axis_index("s")

    def i_start(j, q):
        pltpu.make_async_copy(src4.at[sid, j], srcb.at[q], isems[q]).start()
        pltpu.make_async_copy(dst4.at[sid, j], dstb.at[q], isems[q]).start()

    def i_wait(q):
        pltpu.make_async_copy(src4.at[sid, 0], srcb.at[q], isems[q]).wait()
        pltpu.make_async_copy(dst4.at[sid, 0], dstb.at[q], isems[q]).wait()

    def tfm(q):
        for c in range(BATCH // L):
            v = srcb[q, pl.ds(c * L, L)]
            srcb[q, pl.ds(c * L, L)] = v * 2 + cid

    def g_start(j, q):
        pltpu.make_async_copy(x2.at[srcb.at[q]], rows.at[q], gsems[q]).start()

    def g_wait(q):
        pltpu.make_async_copy(x2.at[srcb.at[0]], rows.at[q], gsems[q]).wait()

    i_start(0, 0)
    i_start(1, 1)
    i_start(2, 2)
    i_wait(0)
    tfm(0)
    g_start(0, 0)
    i_wait(1)
    tfm(1)
    g_start(1, 1)

    zero = jnp.zeros((L,), jnp.float32)

    def zfill(i, _):
        for c in range(DH // L):
            zbuf[i, pl.ds(c * L, L)] = zero
        return 0

    lax.fori_loop(0, ZROWS, zfill, 0)

    def zcopy(k, _):
        pltpu.sync_copy(zbuf, acc.at[pl.ds(sid * ROWS_PER_TILE + k * ZROWS, ZROWS)])
        return 0

    lax.fori_loop(0, ROWS_PER_TILE // ZROWS, zcopy, 0)
    plsc.subcore_barrier()

    def body(jo, _):
        for u in range(NBUF):
            j = jo * NBUF + u
            g_wait(u)

            @pl.when(j + 2 < NB)
            def _():
                q = (u + 2) % NBUF
                i_wait(q)
                tfm(q)
                g_start(j + 2, q)

            @pl.when(j + 3 < NB)
            def _():
                i_start(j + 3, (u + 3) % NBUF)

            pltpu.sync_copy(rows.at[u], acc.at[dstb.at[u]], add=True)
        return 0

    lax.fori_loop(0, NB // NBUF, body, 0)
    plsc.subcore_barrier()

    r0 = sid * ROWS_PER_TILE
    pltpu.sync_copy(acc.at[pl.ds(r0, ROWS_PER_TILE)],
                    out.at[pl.ds(r0, ROWS_PER_TILE), cid])


_mp_kernel = functools.partial(
    pl.kernel,
    mesh=plsc.VectorSubcoreMesh(core_axis_name="c", subcore_axis_name="s"),
    out_type=jax.ShapeDtypeStruct((N_NODES, NC, DH), jnp.float32),
    scratch_types=[
        pltpu.VMEM((NBUF, BATCH), jnp.int32),
        pltpu.VMEM((NBUF, BATCH), jnp.int32),
        pltpu.VMEM((NBUF, BATCH, DH), jnp.float32),
        pltpu.VMEM((ZROWS, DH), jnp.float32),
        pltpu.VMEM_SHARED((N_NODES + PAD_ROWS, DH), jnp.float32),
    ] + [pltpu.SemaphoreType.DMA] * (2 * NBUF),
)(_mp_body)


@jax.jit
def kernel(x, edge_index):
    x2 = x.reshape(N_NODES * 2, DH)
    pad = jnp.arange(E_PAD - N_EDGES, dtype=jnp.int32)
    src4 = jnp.concatenate([edge_index[1], pad % N_NODES]).reshape(NS, NB, BATCH)
    dst4 = jnp.concatenate(
        [edge_index[0], N_NODES + pad % PAD_ROWS]).reshape(NS, NB, BATCH)
    out = _mp_kernel(src4, dst4, x2)
    return out.reshape(N_NODES, D_FEAT)
